# baseline (speedup 1.0000x reference)
import jax
import jax.numpy as jnp
from jax import lax
from jax.experimental import pallas as pl
from jax.experimental.pallas import tpu as pltpu
from jax.experimental.pallas import tpu_sc as plsc

B = 4
S = 2048
C = 2048
POOL = 64
LEN = 5
TOPK = 8
HEAD = TOPK * LEN
CHUNK = 512
JX = S // CHUNK


def _route(sums, pk, sim_ref, idx_ref, idx40_ref, rsum_ref):
    xm = sums * (1.0 / S)
    xn = xm * lax.rsqrt(jnp.maximum(jnp.sum(xm * xm, axis=1, keepdims=True), 1e-12))
    pkn = pk * lax.rsqrt(jnp.maximum(jnp.sum(pk * pk, axis=1, keepdims=True), 1e-12))
    sim = lax.dot_general(
        xn, pkn, (((1,), (1,)), ((), ())),
        preferred_element_type=jnp.float32,
        precision=lax.Precision.HIGHEST,
    )
    sim_ref[...] = sim

    col = lax.broadcasted_iota(jnp.int32, (B, POOL), 1)
    sub = lax.broadcasted_iota(jnp.int32, (B, LEN), 1)
    masked = sim
    acc = jnp.float32(0.0)
    for t in range(TOPK):
        m = jnp.max(masked, axis=1, keepdims=True)
        acc = acc + jnp.sum(m)
        it = jnp.min(jnp.where(masked == m, col, POOL), axis=1)
        idx_ref[:, t : t + 1] = it[:, None]
        idx40_ref[:, LEN * t : LEN * (t + 1)] = it[:, None] * LEN + sub
        masked = jnp.where(col == it[:, None], -jnp.inf, masked)
    rsum_ref[...] = jnp.full((1, 1), acc * (1.0 / B), jnp.float32)


def _bulk_body(x_ref, pk_ref, out_ref, sim_ref, idx_ref, idx40_ref, rsum_ref,
               sums_ref, carry_ref):
    b = pl.program_id(0)
    j = pl.program_id(1)

    @pl.when(j < JX)
    def _():
        part = jnp.sum(x_ref[0], axis=0)[None, None, :]

        @pl.when(j == 0)
        def _():
            sums_ref[pl.ds(b, 1)] = part

        @pl.when(j > 0)
        def _():
            sums_ref[pl.ds(b, 1)] = sums_ref[pl.ds(b, 1)] + part

        out_ref[0, :HEAD, :] = carry_ref[...]
        out_ref[0, HEAD:, :] = x_ref[0, : CHUNK - HEAD, :]
        carry_ref[...] = x_ref[0, CHUNK - HEAD :, :]

    @pl.when(j == JX)
    def _():
        out_ref[0, :HEAD, :] = carry_ref[...]

    @pl.when((b == B - 1) & (j == JX))
    def _():
        _route(sums_ref[:, 0, :], pk_ref[...], sim_ref, idx_ref, idx40_ref,
               rsum_ref)


def _head_body(idx40_hbm, prompt_hbm, out_hbm, idx_v, rows_v, sem):
    wid = lax.axis_index("s") * 2 + lax.axis_index("c")

    @pl.when(wid < B)
    def _():
        b = wid
        pltpu.sync_copy(idx40_hbm.at[b], idx_v)
        pltpu.async_copy(prompt_hbm.at[idx_v], rows_v, sem).wait()
        pltpu.sync_copy(rows_v, out_hbm.at[b, pl.ds(0, HEAD), :])


def kernel(x_embed, prompt, prompt_key):
    big, sim, idx, idx40, rsum = pl.pallas_call(
        _bulk_body,
        grid=(B, JX + 1),
        in_specs=[
            pl.BlockSpec((1, CHUNK, C), lambda b, j: (b, jnp.minimum(j, JX - 1), 0)),
            pl.BlockSpec((POOL, C), lambda b, j: (0, 0)),
        ],
        out_specs=[
            pl.BlockSpec((1, CHUNK, C), lambda b, j: (b, j, 0)),
            pl.BlockSpec((B, POOL), lambda b, j: (0, 0)),
            pl.BlockSpec((B, TOPK), lambda b, j: (0, 0)),
            pl.BlockSpec((B, HEAD), lambda b, j: (0, 0)),
            pl.BlockSpec((1, 1), lambda b, j: (0, 0)),
        ],
        out_shape=[
            jax.ShapeDtypeStruct((B, HEAD + S, C), jnp.float32),
            jax.ShapeDtypeStruct((B, POOL), jnp.float32),
            jax.ShapeDtypeStruct((B, TOPK), jnp.int32),
            jax.ShapeDtypeStruct((B, HEAD), jnp.int32),
            jax.ShapeDtypeStruct((1, 1), jnp.float32),
        ],
        scratch_shapes=[
            pltpu.VMEM((B, 1, C), jnp.float32),
            pltpu.VMEM((HEAD, C), jnp.float32),
        ],
    )(x_embed, prompt_key)

    mesh = plsc.VectorSubcoreMesh(core_axis_name="c", subcore_axis_name="s")
    gather_head = pl.kernel(
        _head_body,
        out_type=(),
        mesh=mesh,
        scratch_types=[
            pltpu.VMEM((HEAD,), jnp.int32),
            pltpu.VMEM((HEAD, C), jnp.float32),
            pltpu.SemaphoreType.DMA,
        ],
    )
    big_ref = jax.new_ref(big)
    gather_head(idx40, prompt.reshape(POOL * LEN, C), big_ref)
    prompted = jax.freeze(big_ref)

    return (prompted, rsum[0, 0], sim, idx)
